# Initial kernel scaffold; baseline (speedup 1.0000x reference)
import jax
import jax.numpy as jnp
from jax.experimental import pallas as pl


def kernel(image, rgb2yuv):
    raise NotImplementedError("write your pallas kernel here")


# speedup vs baseline: 11.0140x; 11.0140x over previous
import math

import jax
import jax.numpy as jnp
from jax.experimental import pallas as pl
from jax.experimental.pallas import tpu as pltpu

_BLKPT = 1.0
_WHTPT = 99.0
_MAX_MULT = 1.5

_ROWS = 128
_NB1 = 8
_BITSPAN = 1 << 30
_IMAX = 2**31 - 1


def _y_hist_kernel(img_ref, m_ref, y_ref, cnt_ref):
    c = pl.program_id(1)
    r = img_ref[0, 0]
    g = img_ref[0, 1]
    b = img_ref[0, 2]
    y = r * m_ref[0, 0] + g * m_ref[0, 1] + b * m_ref[0, 2]
    bits = jax.lax.bitcast_convert_type(y, jnp.int32)
    y_ref[0] = bits

    @pl.when(c == 0)
    def _():
        cnt_ref[...] = jnp.zeros_like(cnt_ref)

    for j in range(1, _NB1):
        t = jnp.int32(j * (_BITSPAN // _NB1))
        psum = jnp.sum((bits >= t).astype(jnp.int32), axis=0)
        cnt_ref[0, j - 1, :] += psum


def _count_kernel(y_ref, bnd_ref, cnt_ref):
    i = pl.program_id(0)
    c = pl.program_id(1)
    bits = y_ref[0]

    @pl.when(c == 0)
    def _():
        cnt_ref[...] = jnp.zeros_like(cnt_ref)

    for r in range(2):
        for j in range(3):
            t = bnd_ref[i, r, j]
            psum = jnp.sum((bits >= t).astype(jnp.int32), axis=0)
            cnt_ref[0, r * 3 + j, :] += psum


def _finish_kernel(y_ref, v_ref, cnt_ref, mn_ref):
    i = pl.program_id(0)
    c = pl.program_id(1)
    bits = y_ref[0]

    @pl.when(c == 0)
    def _():
        cnt_ref[...] = jnp.zeros_like(cnt_ref)
        mn_ref[...] = jnp.full_like(mn_ref, _IMAX)

    for r in range(2):
        v = v_ref[i, r]
        le = bits <= v
        cnt_ref[0, r, :] += jnp.sum(le.astype(jnp.int32), axis=0)
        above = jnp.where(le, _IMAX, bits)
        mn_ref[0, r, :] = jnp.minimum(mn_ref[0, r, :], jnp.min(above, axis=0))


def _apply_kernel(img_ref, blk_ref, mul_ref, out_ref):
    i = pl.program_id(0)
    bp = blk_ref[i]
    mu = mul_ref[i]
    out_ref[...] = jnp.clip((img_ref[...] - bp) * mu, 0.0, 1.0)


def kernel(image, rgb2yuv):
    B, C, H, W = image.shape
    n = H * W
    rows = _ROWS if H % _ROWS == 0 else H
    nc = H // rows
    grid = (B, nc)
    dims = ("parallel", "arbitrary")

    y_bits, pcnt0 = pl.pallas_call(
        _y_hist_kernel,
        grid=grid,
        in_specs=[
            pl.BlockSpec((1, C, rows, W), lambda b, c: (b, 0, c, 0)),
            pl.BlockSpec(memory_space=pltpu.SMEM),
        ],
        out_specs=[
            pl.BlockSpec((1, rows, W), lambda b, c: (b, c, 0)),
            pl.BlockSpec((1, _NB1 - 1, W), lambda b, c: (b, 0, 0)),
        ],
        out_shape=[
            jax.ShapeDtypeStruct((B, H, W), jnp.int32),
            jax.ShapeDtypeStruct((B, _NB1 - 1, W), jnp.int32),
        ],
        compiler_params=pltpu.CompilerParams(dimension_semantics=dims),
    )(image, rgb2yuv)
    cnt0 = jnp.sum(pcnt0, axis=-1)

    pos_b = _BLKPT / 100.0 * (n - 1)
    pos_w = _WHTPT / 100.0 * (n - 1)
    rank_b = int(math.floor(pos_b))
    rank_w = int(math.floor(pos_w))
    frac_b = jnp.float32(pos_b - rank_b)
    frac_w = jnp.float32(pos_w - rank_w)
    ranks = jnp.array([rank_b, rank_w], jnp.int32)

    clt0 = n - cnt0
    k0 = jnp.sum(clt0[:, None, :] <= ranks[None, :, None], axis=-1)
    k0 = k0.astype(jnp.int32)
    w0 = _BITSPAN // _NB1
    lo = k0 * w0
    width = jnp.full_like(lo, w0)

    count_call = pl.pallas_call(
        _count_kernel,
        grid=grid,
        in_specs=[
            pl.BlockSpec((1, rows, W), lambda b, c: (b, c, 0)),
            pl.BlockSpec(memory_space=pltpu.SMEM),
        ],
        out_specs=pl.BlockSpec((1, 6, W), lambda b, c: (b, 0, 0)),
        out_shape=jax.ShapeDtypeStruct((B, 6, W), jnp.int32),
        compiler_params=pltpu.CompilerParams(dimension_semantics=dims),
    )

    wbound = w0
    while wbound > 1:
        j = jnp.arange(1, 4, dtype=jnp.int32)
        edges = (j[None, None, :] * width[:, :, None]) // 4
        bnd = lo[:, :, None] + edges
        pc = count_call(y_bits, bnd)
        cge = jnp.sum(pc, axis=-1).reshape(B, 2, 3)
        clt = n - cge
        k = jnp.sum(clt <= ranks[None, :, None], axis=-1).astype(jnp.int32)
        e_lo = (k * width) // 4
        e_hi = ((k + 1) * width) // 4
        lo = lo + e_lo
        width = e_hi - e_lo
        wbound = -(-wbound // 4)

    pcnt_le, pmn_above = pl.pallas_call(
        _finish_kernel,
        grid=grid,
        in_specs=[
            pl.BlockSpec((1, rows, W), lambda b, c: (b, c, 0)),
            pl.BlockSpec(memory_space=pltpu.SMEM),
        ],
        out_specs=[
            pl.BlockSpec((1, 2, W), lambda b, c: (b, 0, 0)),
            pl.BlockSpec((1, 2, W), lambda b, c: (b, 0, 0)),
        ],
        out_shape=[
            jax.ShapeDtypeStruct((B, 2, W), jnp.int32),
            jax.ShapeDtypeStruct((B, 2, W), jnp.int32),
        ],
        compiler_params=pltpu.CompilerParams(dimension_semantics=dims),
    )(y_bits, lo)

    cnt_le = jnp.sum(pcnt_le, axis=-1)
    mn_above = jnp.min(pmn_above, axis=-1)
    s_lo = jax.lax.bitcast_convert_type(lo, jnp.float32)
    s_up = jax.lax.bitcast_convert_type(mn_above, jnp.float32)
    nxt = jnp.where(cnt_le >= ranks[None, :] + 2, s_lo, s_up)
    blkpt = s_lo[:, 0] * (1.0 - frac_b) + nxt[:, 0] * frac_b
    whtpt = s_lo[:, 1] * (1.0 - frac_w) + nxt[:, 1] * frac_w
    mult = jnp.minimum(1.0 / (whtpt - blkpt), _MAX_MULT)

    return pl.pallas_call(
        _apply_kernel,
        grid=grid,
        in_specs=[
            pl.BlockSpec((1, C, rows, W), lambda b, c: (b, 0, c, 0)),
            pl.BlockSpec(memory_space=pltpu.SMEM),
            pl.BlockSpec(memory_space=pltpu.SMEM),
        ],
        out_specs=pl.BlockSpec((1, C, rows, W), lambda b, c: (b, 0, c, 0)),
        out_shape=jax.ShapeDtypeStruct((B, C, H, W), jnp.float32),
        compiler_params=pltpu.CompilerParams(dimension_semantics=dims),
    )(image, blkpt, mult)
